# baseline (speedup 1.0000x reference)
import functools

import jax
import jax.numpy as jnp
from jax import lax
from jax.experimental import pallas as pl
from jax.experimental.pallas import tpu as pltpu
from jax.experimental.pallas import tpu_sc as plsc

RB = 16
NBUF = 2


def kernel(x, emb):
    b, s, d = x.shape
    rows = b * s
    xr = x.reshape(rows, d)

    info = plsc.get_sparse_core_info()
    nc, ns = info.num_cores, info.num_subcores
    nw = nc * ns
    rows_per_w = rows // nw
    nblocks = rows_per_w // RB
    ngroups = nblocks // NBUF

    mesh = plsc.VectorSubcoreMesh(core_axis_name="c", subcore_axis_name="s")

    @functools.partial(
        pl.kernel,
        mesh=mesh,
        out_type=jax.ShapeDtypeStruct((rows, d), jnp.float32),
        scratch_types=[
            pltpu.VMEM((NBUF, RB, d), jnp.float32),
            pltpu.VMEM((NBUF, RB, d), jnp.float32),
            pltpu.VMEM((NBUF, RB, d), jnp.float32),
            pltpu.SemaphoreType.DMA((NBUF,)),
            pltpu.SemaphoreType.DMA((NBUF,)),
            pltpu.SemaphoreType.DMA((NBUF,)),
        ],
    )
    def sc_add(x_hbm, e_hbm, o_hbm, bx, be, bo, semx, seme, semo):
        wid = lax.axis_index("s") * nc + lax.axis_index("c")
        r0 = wid * rows_per_w

        def start_in(g, k):
            rbase = r0 + g * RB
            bb = rbase // s
            sbase = rbase - bb * s
            pltpu.make_async_copy(
                x_hbm.at[pl.ds(rbase, RB)], bx.at[k], semx.at[k]
            ).start()
            pltpu.make_async_copy(
                e_hbm.at[pl.ds(sbase, RB)], be.at[k], seme.at[k]
            ).start()

        for k in range(NBUF):
            start_in(k, k)

        def group(gg, carry):
            for k in range(NBUF):
                g = gg * NBUF + k
                rbase = r0 + g * RB
                pltpu.make_async_copy(
                    x_hbm.at[pl.ds(rbase, RB)], bx.at[k], semx.at[k]
                ).wait()
                bb = rbase // s
                sbase = rbase - bb * s
                pltpu.make_async_copy(
                    e_hbm.at[pl.ds(sbase, RB)], be.at[k], seme.at[k]
                ).wait()

                @pl.when(gg > 0)
                def _drain():
                    pltpu.make_async_copy(
                        bo.at[k], o_hbm.at[pl.ds(rbase - NBUF * RB, RB)], semo.at[k]
                    ).wait()

                def row(i, c):
                    for j in range(0):
                        sl = pl.ds(j * 16, 16)
                        bo[k, i, sl] = bx[k, i, sl] + be[k, i, sl]
                    return c

                lax.fori_loop(0, RB, row, 0)

                pltpu.make_async_copy(
                    bo.at[k], o_hbm.at[pl.ds(rbase, RB)], semo.at[k]
                ).start()

                @pl.when(gg < ngroups - 1)
                def _prefetch():
                    start_in(g + NBUF, k)

            return carry

        lax.fori_loop(0, ngroups, group, 0)

        for k in range(NBUF):
            rbase = r0 + (ngroups - 1) * NBUF * RB + k * RB
            pltpu.make_async_copy(
                bo.at[k], o_hbm.at[pl.ds(rbase, RB)], semo.at[k]
            ).wait()

    out = sc_add(xr, emb)
    return out.reshape(b, s, d)


# speedup vs baseline: 1.6838x; 1.6838x over previous
import jax
import jax.numpy as jnp
from jax.experimental import pallas as pl

BLK_S = 1024
BLK_B = 2


def _add_kernel(x_ref, e_ref, o_ref):
    o_ref[...] = x_ref[...] + e_ref[...][None, :, :]


def kernel(x, emb):
    b, s, d = x.shape
    grid = (s // BLK_S, b // BLK_B)
    return pl.pallas_call(
        _add_kernel,
        grid=grid,
        in_specs=[
            pl.BlockSpec((BLK_B, BLK_S, d), lambda i, j: (j, i, 0)),
            pl.BlockSpec((BLK_S, d), lambda i, j: (i, 0)),
        ],
        out_specs=pl.BlockSpec((BLK_B, BLK_S, d), lambda i, j: (j, i, 0)),
        out_shape=jax.ShapeDtypeStruct((b, s, d), x.dtype),
    )(x, emb)
